# baseline (speedup 1.0000x reference)
import jax
import jax.numpy as jnp
from jax import lax
from jax.experimental import pallas as pl
from jax.experimental.pallas import tpu as pltpu
from jax.experimental.pallas import tpu_sc as plsc

N_NODES = 50000
DIM = 64
N_EDGES = 800000

NC, NS, LANES = 2, 16, 16
NPAD = 50176
HALF = NPAD // 2
NODES_PER_TILE = HALF // NS
BATCH = 64
GROUP = 14
N_GROUPS = 56
NB = GROUP * N_GROUPS
EDGES_PER_TILE = NB * BATCH
EPAD = EDGES_PER_TILE * NS
ACC_ROWS = HALF + 8
DEG_SIZE = NPAD + 128
DEG_PER_TILE = DEG_SIZE // NS
NCHUNK = 112
N_NCHUNKS = NODES_PER_TILE // NCHUNK
RING = 7


def _mesh():
    return plsc.VectorSubcoreMesh(core_axis_name="c", subcore_axis_name="s")


def _rsqrt_newton(x):
    s = 0.5 * (x + 1.0)
    for _ in range(20):
        s = 0.5 * (s + x / s)
    return 1.0 / s


def _zero_nodebuf(nodebuf):
    @pl.loop(0, NCHUNK)
    def _(i):
        for q in range(DIM // LANES):
            nodebuf[i, pl.ds(q * LANES, LANES)] = jnp.zeros((LANES,), jnp.float32)


def _scale_nodes_chunk(nodebuf, disbuf, square):
    @pl.loop(0, NCHUNK // LANES)
    def _(ii):
        dvec = disbuf[pl.ds(ii * LANES, LANES)]
        if square:
            dvec = dvec * dvec
        for r in range(LANES):
            d = dvec[r]
            row = ii * LANES + r
            for q in range(DIM // LANES):
                sl = pl.ds(q * LANES, LANES)
                nodebuf[row, sl] = nodebuf[row, sl] * d


def _prep_body(col_hbm, emb_hbm, dis_out, t0_out, loc_out,
               deg_sp, colbuf, locbuf, zbuf, degv, disv, nodebuf, ones_v, sem):
    c = lax.axis_index("c")
    s = lax.axis_index("s")

    @pl.loop(0, DEG_PER_TILE // LANES)
    def _(i):
        zbuf[pl.ds(i * LANES, LANES)] = jnp.zeros((LANES,), jnp.float32)

    pltpu.sync_copy(zbuf, deg_sp.at[pl.ds(s * DEG_PER_TILE, DEG_PER_TILE)])

    @pl.loop(0, BATCH // LANES)
    def _(i):
        ones_v[pl.ds(i * LANES, LANES)] = jnp.ones((LANES,), jnp.float32)

    plsc.subcore_barrier()

    base_lo = c * HALF
    @pl.loop(0, N_GROUPS)
    def _(g):
        pltpu.sync_copy(col_hbm.at[s, pl.ds(g * GROUP, GROUP)], colbuf)

        @pl.loop(0, GROUP)
        def _(j):
            for m in range(BATCH // LANES):
                sl = pl.ds(m * LANES, LANES)
                v = colbuf[j, sl]
                loc = v - base_lo
                valid = (loc >= 0) & (loc < HALF)
                locbuf[j, sl] = jnp.where(valid, loc, HALF)

        pltpu.sync_copy(locbuf, loc_out.at[c, s, pl.ds(g * GROUP, GROUP)])

        @pl.loop(0, GROUP)
        def _(j):
            pltpu.async_copy(ones_v, deg_sp.at[colbuf.at[j]], sem, add=True)

        @pl.loop(0, GROUP)
        def _(j):
            pltpu.make_async_copy(ones_v, deg_sp.at[colbuf.at[j]], sem).wait()

    plsc.subcore_barrier()

    lo_local = s * NODES_PER_TILE
    lo_glob = base_lo + lo_local
    pltpu.sync_copy(deg_sp.at[pl.ds(lo_glob, NODES_PER_TILE)], degv)

    @pl.loop(0, NODES_PER_TILE // LANES)
    def _(i):
        sl = pl.ds(i * LANES, LANES)
        x = degv[sl]
        y = _rsqrt_newton(x)
        disv[sl] = jnp.where(x > 0.0, y, 0.0)

    pltpu.sync_copy(disv, dis_out.at[pl.ds(lo_glob, NODES_PER_TILE)])

    for k in range(N_NCHUNKS):
        gbase = lo_glob + k * NCHUNK
        pltpu.sync_copy(emb_hbm.at[pl.ds(gbase, NCHUNK)], nodebuf)
        _scale_nodes_chunk(nodebuf, disv.at[pl.ds(k * NCHUNK, NCHUNK)],
                           square=False)
        pltpu.sync_copy(nodebuf, t0_out.at[pl.ds(gbase, NCHUNK)])


def _layer_body(t_in, row_hbm, loc_hbm, dis_hbm, t_out,
                acc, gbuf, rowbuf, locbuf, disbuf, gsems, ssems):
    c = lax.axis_index("c")
    s = lax.axis_index("s")

    nodebuf = gbuf.at[0].at[pl.ds(0, NCHUNK)]

    _zero_nodebuf(nodebuf)
    lo_local = s * NODES_PER_TILE
    for k in range(N_NCHUNKS):
        pltpu.sync_copy(nodebuf, acc.at[pl.ds(lo_local + k * NCHUNK, NCHUNK)])

    plsc.subcore_barrier()

    @pl.loop(0, N_GROUPS)
    def _(g):
        pltpu.sync_copy(row_hbm.at[s, pl.ds(g * GROUP, GROUP)], rowbuf)
        pltpu.sync_copy(loc_hbm.at[c, s, pl.ds(g * GROUP, GROUP)], locbuf)

        @pl.loop(0, GROUP // RING)
        def _(q):
            for b in range(RING):
                j = q * RING + b

                @pl.when(q > 0)
                def _():
                    pltpu.make_async_copy(gbuf.at[b], acc.at[locbuf.at[j]],
                                          ssems[b]).wait()

                pltpu.async_copy(t_in.at[rowbuf.at[j]], gbuf.at[b], gsems[b])

            for b in range(RING):
                j = q * RING + b
                pltpu.make_async_copy(t_in.at[rowbuf.at[j]], gbuf.at[b],
                                      gsems[b]).wait()
                pltpu.async_copy(gbuf.at[b], acc.at[locbuf.at[j]], ssems[b],
                                 add=True)

        for b in range(RING):
            j = GROUP - RING + b
            pltpu.make_async_copy(gbuf.at[b], acc.at[locbuf.at[j]],
                                  ssems[b]).wait()

    plsc.subcore_barrier()

    base_lo = c * HALF
    for k in range(N_NCHUNKS):
        lo = lo_local + k * NCHUNK
        gbase = base_lo + lo
        pltpu.sync_copy(acc.at[pl.ds(lo, NCHUNK)], nodebuf)
        pltpu.sync_copy(dis_hbm.at[pl.ds(gbase, NCHUNK)], disbuf)
        _scale_nodes_chunk(nodebuf, disbuf, square=True)
        pltpu.sync_copy(nodebuf, t_out.at[pl.ds(gbase, NCHUNK)])


@jax.jit
def _prep(col_t, emb_p):
    return pl.kernel(
        _prep_body,
        out_type=(
            jax.ShapeDtypeStruct((NPAD,), jnp.float32),
            jax.ShapeDtypeStruct((NPAD, DIM), jnp.float32),
            jax.ShapeDtypeStruct((NC, NS, NB, BATCH), jnp.int32),
        ),
        mesh=_mesh(),
        compiler_params=pltpu.CompilerParams(use_tc_tiling_on_sc=False),
        scratch_types=[
            pltpu.VMEM_SHARED((DEG_SIZE,), jnp.float32),
            pltpu.VMEM((GROUP, BATCH), jnp.int32),
            pltpu.VMEM((GROUP, BATCH), jnp.int32),
            pltpu.VMEM((DEG_PER_TILE,), jnp.float32),
            pltpu.VMEM((NODES_PER_TILE,), jnp.float32),
            pltpu.VMEM((NODES_PER_TILE,), jnp.float32),
            pltpu.VMEM((NCHUNK, DIM), jnp.float32),
            pltpu.VMEM((BATCH,), jnp.float32),
            pltpu.SemaphoreType.DMA,
        ],
    )(col_t, emb_p)


@jax.jit
def _layer(t_in, row_t, loc, dis):
    return pl.kernel(
        _layer_body,
        out_type=jax.ShapeDtypeStruct((NPAD, DIM), jnp.float32),
        mesh=_mesh(),
        compiler_params=pltpu.CompilerParams(use_tc_tiling_on_sc=False),
        scratch_types=[
            pltpu.VMEM_SHARED((ACC_ROWS, DIM), jnp.float32),
            pltpu.VMEM((RING, BATCH, DIM), jnp.float32),
            pltpu.VMEM((GROUP, BATCH), jnp.int32),
            pltpu.VMEM((GROUP, BATCH), jnp.int32),
            pltpu.VMEM((NCHUNK,), jnp.float32),
            [pltpu.SemaphoreType.DMA] * RING,
            [pltpu.SemaphoreType.DMA] * RING,
        ],
    )(t_in, row_t, loc, dis)


def _mean_body(emb_ref, t1_ref, t2_ref, t3_ref, dis_ref, out_ref):
    d = dis_ref[...]
    inv = jnp.where(d > 0.0, 1.0 / jnp.where(d > 0.0, d, 1.0), 0.0)
    ts = t1_ref[...] + t2_ref[...] + t3_ref[...]
    out_ref[...] = (emb_ref[...] + ts * inv) * 0.25


@jax.jit
def _mean(emb_p, t1, t2, t3, dis2d):
    blk = 512
    grid = NPAD // blk
    return pl.pallas_call(
        _mean_body,
        out_shape=jax.ShapeDtypeStruct((NPAD, DIM), jnp.float32),
        grid=(grid,),
        in_specs=[
            pl.BlockSpec((blk, DIM), lambda i: (i, 0)),
            pl.BlockSpec((blk, DIM), lambda i: (i, 0)),
            pl.BlockSpec((blk, DIM), lambda i: (i, 0)),
            pl.BlockSpec((blk, DIM), lambda i: (i, 0)),
            pl.BlockSpec((blk, 1), lambda i: (i, 0)),
        ],
        out_specs=pl.BlockSpec((blk, DIM), lambda i: (i, 0)),
    )(emb_p, t1, t2, t3, dis2d)


def kernel(emb_weight, edge_index):
    ei = edge_index.astype(jnp.int32)
    pad = EPAD - N_EDGES
    row = jnp.concatenate([ei[0], jnp.zeros((pad,), jnp.int32)])
    col = jnp.concatenate([ei[1], jnp.full((pad,), NPAD, jnp.int32)])
    row_t = row.reshape(NS, NB, BATCH)
    col_t = col.reshape(NS, NB, BATCH)
    emb_p = jnp.pad(emb_weight, ((0, NPAD - N_NODES), (0, 0)))

    dis, t0, loc = _prep(col_t, emb_p)
    t1 = _layer(t0, row_t, loc, dis)
    t2 = _layer(t1, row_t, loc, dis)
    t3 = _layer(t2, row_t, loc, dis)
    out = _mean(emb_p, t1, t2, t3, dis.reshape(NPAD, 1))
    return out[:N_NODES]


# speedup vs baseline: 1.2130x; 1.2130x over previous
import jax
import jax.numpy as jnp
from jax import lax
from jax.experimental import pallas as pl
from jax.experimental.pallas import tpu as pltpu
from jax.experimental.pallas import tpu_sc as plsc

N_NODES = 50000
DIM = 64
N_EDGES = 800000

NC, NS, LANES = 2, 16, 16
NPAD = 50176
HALF = NPAD // 2
NODES_PER_TILE = HALF // NS
BATCH = 64
GROUP = 14
N_GROUPS = 56
NB = GROUP * N_GROUPS
EDGES_PER_TILE = NB * BATCH
EPAD = EDGES_PER_TILE * NS
CAP = EDGES_PER_TILE
ACC_ROWS = HALF + 8
DEG_SIZE = NPAD + 128
DEG_PER_TILE = DEG_SIZE // NS
NCHUNK = 112
N_NCHUNKS = NODES_PER_TILE // NCHUNK
RING = 7
EDGES_PER_GROUP = GROUP * BATCH


def _mesh():
    return plsc.VectorSubcoreMesh(core_axis_name="c", subcore_axis_name="s")


def _rsqrt_newton(x):
    s = 0.5 * (x + 1.0)
    for _ in range(20):
        s = 0.5 * (s + x / s)
    return 1.0 / s


def _zero_nodebuf(nodebuf):
    @pl.loop(0, NCHUNK)
    def _(i):
        for q in range(DIM // LANES):
            nodebuf[i, pl.ds(q * LANES, LANES)] = jnp.zeros((LANES,), jnp.float32)


def _scale_nodes_chunk(nodebuf, disbuf, square):
    @pl.loop(0, NCHUNK // LANES)
    def _(ii):
        dvec = disbuf[pl.ds(ii * LANES, LANES)]
        if square:
            dvec = dvec * dvec
        for r in range(LANES):
            d = dvec[r]
            row = ii * LANES + r
            for q in range(DIM // LANES):
                sl = pl.ds(q * LANES, LANES)
                nodebuf[row, sl] = nodebuf[row, sl] * d


def _prep_body(col_hbm, row_hbm, emb_hbm, dis_out, t0_out, rowc_out, locc_out,
               cnt_out, deg_sp, colbuf, rowgrp, crow, cloc, zbuf, degv, disv,
               nodebuf, ones_v, cntv, sem):
    c = lax.axis_index("c")
    s = lax.axis_index("s")

    @pl.loop(0, DEG_PER_TILE // LANES)
    def _(i):
        zbuf[pl.ds(i * LANES, LANES)] = jnp.zeros((LANES,), jnp.float32)

    pltpu.sync_copy(zbuf, deg_sp.at[pl.ds(s * DEG_PER_TILE, DEG_PER_TILE)])

    @pl.loop(0, BATCH // LANES)
    def _(i):
        ones_v[pl.ds(i * LANES, LANES)] = jnp.ones((LANES,), jnp.float32)

    @pl.loop(0, CAP // LANES)
    def _(k):
        sl = pl.ds(k * LANES, LANES)
        crow[sl] = jnp.zeros((LANES,), jnp.int32)
        cloc[sl] = jnp.full((LANES,), HALF, jnp.int32)

    plsc.subcore_barrier()

    base_lo = c * HALF
    def _group(g, ptr):
        pltpu.sync_copy(col_hbm.at[s, pl.ds(g * GROUP, GROUP)], colbuf)
        pltpu.sync_copy(row_hbm.at[s, pl.ds(g * GROUP, GROUP)], rowgrp)

        @pl.loop(0, GROUP)
        def _(j):
            pltpu.async_copy(ones_v, deg_sp.at[colbuf.at[j]], sem, add=True)

        def _vreg(t, p):
            j = t // (BATCH // LANES)
            m = t % (BATCH // LANES)
            sl = pl.ds(m * LANES, LANES)
            vc = colbuf[j, sl]
            vr = rowgrp[j, sl]
            loc = vc - base_lo
            valid = (loc >= 0) & (loc < HALF)
            pos = plsc.cumsum(jnp.where(valid, jnp.int32(1), jnp.int32(0)))
            idx = jnp.where(valid, p + pos - 1, jnp.int32(CAP))
            plsc.store_scatter(crow, [idx], vr)
            plsc.store_scatter(cloc, [idx], jnp.where(valid, loc, HALF))
            return p + pos[LANES - 1]

        ptr = pl.loop(0, GROUP * (BATCH // LANES), init_carry=ptr)(_vreg)

        @pl.loop(0, GROUP)
        def _(j):
            pltpu.make_async_copy(ones_v, deg_sp.at[colbuf.at[j]], sem).wait()

        return ptr

    cnt = pl.loop(0, N_GROUPS, init_carry=jnp.int32(0))(_group)

    pltpu.sync_copy(crow.at[pl.ds(0, CAP)], rowc_out.at[c, s])
    pltpu.sync_copy(cloc.at[pl.ds(0, CAP)], locc_out.at[c, s])
    cntv[...] = jnp.full((LANES,), cnt, jnp.int32)
    pltpu.sync_copy(cntv, cnt_out.at[c, s])

    plsc.subcore_barrier()

    lo_local = s * NODES_PER_TILE
    lo_glob = base_lo + lo_local
    pltpu.sync_copy(deg_sp.at[pl.ds(lo_glob, NODES_PER_TILE)], degv)

    @pl.loop(0, NODES_PER_TILE // LANES)
    def _(i):
        sl = pl.ds(i * LANES, LANES)
        x = degv[sl]
        y = _rsqrt_newton(x)
        disv[sl] = jnp.where(x > 0.0, y, 0.0)

    pltpu.sync_copy(disv, dis_out.at[pl.ds(lo_glob, NODES_PER_TILE)])

    for k in range(N_NCHUNKS):
        gbase = lo_glob + k * NCHUNK
        pltpu.sync_copy(emb_hbm.at[pl.ds(gbase, NCHUNK)], nodebuf)
        _scale_nodes_chunk(nodebuf, disv.at[pl.ds(k * NCHUNK, NCHUNK)],
                           square=False)
        pltpu.sync_copy(nodebuf, t0_out.at[pl.ds(gbase, NCHUNK)])


def _layer_body(t_in, rowc, locc, cnt_in, dis_hbm, t_out,
                acc, gbuf, rowbuf, locbuf, disbuf, cntv, gsems, ssems):
    c = lax.axis_index("c")
    s = lax.axis_index("s")

    nodebuf = gbuf.at[0].at[pl.ds(0, NCHUNK)]

    _zero_nodebuf(nodebuf)
    lo_local = s * NODES_PER_TILE
    for k in range(N_NCHUNKS):
        pltpu.sync_copy(nodebuf, acc.at[pl.ds(lo_local + k * NCHUNK, NCHUNK)])

    plsc.subcore_barrier()

    pltpu.sync_copy(cnt_in.at[c, s], cntv)
    cnt = cntv[...][0]
    nq = lax.div(cnt + (EDGES_PER_GROUP - 1), jnp.int32(EDGES_PER_GROUP))

    @pl.loop(0, nq)
    def _(g):
        pltpu.sync_copy(rowc.at[c, s, pl.ds(g * GROUP, GROUP)], rowbuf)
        pltpu.sync_copy(locc.at[c, s, pl.ds(g * GROUP, GROUP)], locbuf)

        @pl.loop(0, GROUP // RING)
        def _(q):
            for b in range(RING):
                j = q * RING + b

                @pl.when(q > 0)
                def _():
                    pltpu.make_async_copy(gbuf.at[b], acc.at[locbuf.at[j]],
                                          ssems[b]).wait()

                pltpu.async_copy(t_in.at[rowbuf.at[j]], gbuf.at[b], gsems[b])

            for b in range(RING):
                j = q * RING + b
                pltpu.make_async_copy(t_in.at[rowbuf.at[j]], gbuf.at[b],
                                      gsems[b]).wait()
                pltpu.async_copy(gbuf.at[b], acc.at[locbuf.at[j]], ssems[b],
                                 add=True)

        for b in range(RING):
            j = GROUP - RING + b
            pltpu.make_async_copy(gbuf.at[b], acc.at[locbuf.at[j]],
                                  ssems[b]).wait()

    plsc.subcore_barrier()

    base_lo = c * HALF
    for k in range(N_NCHUNKS):
        lo = lo_local + k * NCHUNK
        gbase = base_lo + lo
        pltpu.sync_copy(acc.at[pl.ds(lo, NCHUNK)], nodebuf)
        pltpu.sync_copy(dis_hbm.at[pl.ds(gbase, NCHUNK)], disbuf)
        _scale_nodes_chunk(nodebuf, disbuf, square=True)
        pltpu.sync_copy(nodebuf, t_out.at[pl.ds(gbase, NCHUNK)])


@jax.jit
def _prep(col_t, row_t, emb_p):
    return pl.kernel(
        _prep_body,
        out_type=(
            jax.ShapeDtypeStruct((NPAD,), jnp.float32),
            jax.ShapeDtypeStruct((NPAD, DIM), jnp.float32),
            jax.ShapeDtypeStruct((NC, NS, CAP), jnp.int32),
            jax.ShapeDtypeStruct((NC, NS, CAP), jnp.int32),
            jax.ShapeDtypeStruct((NC, NS, LANES), jnp.int32),
        ),
        mesh=_mesh(),
        compiler_params=pltpu.CompilerParams(use_tc_tiling_on_sc=False, needs_layout_passes=False),
        scratch_types=[
            pltpu.VMEM_SHARED((DEG_SIZE,), jnp.float32),
            pltpu.VMEM((GROUP, BATCH), jnp.int32),
            pltpu.VMEM((GROUP, BATCH), jnp.int32),
            pltpu.VMEM((CAP + LANES,), jnp.int32),
            pltpu.VMEM((CAP + LANES,), jnp.int32),
            pltpu.VMEM((DEG_PER_TILE,), jnp.float32),
            pltpu.VMEM((NODES_PER_TILE,), jnp.float32),
            pltpu.VMEM((NODES_PER_TILE,), jnp.float32),
            pltpu.VMEM((NCHUNK, DIM), jnp.float32),
            pltpu.VMEM((BATCH,), jnp.float32),
            pltpu.VMEM((LANES,), jnp.int32),
            pltpu.SemaphoreType.DMA,
        ],
    )(col_t, row_t, emb_p)


@jax.jit
def _layer(t_in, rowc4, locc4, cnts, dis):
    return pl.kernel(
        _layer_body,
        out_type=jax.ShapeDtypeStruct((NPAD, DIM), jnp.float32),
        mesh=_mesh(),
        compiler_params=pltpu.CompilerParams(use_tc_tiling_on_sc=False, needs_layout_passes=False),
        scratch_types=[
            pltpu.VMEM_SHARED((ACC_ROWS, DIM), jnp.float32),
            pltpu.VMEM((RING, BATCH, DIM), jnp.float32),
            pltpu.VMEM((GROUP, BATCH), jnp.int32),
            pltpu.VMEM((GROUP, BATCH), jnp.int32),
            pltpu.VMEM((NCHUNK,), jnp.float32),
            pltpu.VMEM((LANES,), jnp.int32),
            [pltpu.SemaphoreType.DMA] * RING,
            [pltpu.SemaphoreType.DMA] * RING,
        ],
    )(t_in, rowc4, locc4, cnts, dis)


def _mean_body(emb_ref, t1_ref, t2_ref, t3_ref, dis_ref, out_ref):
    d = dis_ref[...]
    inv = jnp.where(d > 0.0, 1.0 / jnp.where(d > 0.0, d, 1.0), 0.0)
    ts = t1_ref[...] + t2_ref[...] + t3_ref[...]
    out_ref[...] = (emb_ref[...] + ts * inv) * 0.25


@jax.jit
def _mean(emb_p, t1, t2, t3, dis2d):
    blk = 512
    grid = NPAD // blk
    return pl.pallas_call(
        _mean_body,
        out_shape=jax.ShapeDtypeStruct((NPAD, DIM), jnp.float32),
        grid=(grid,),
        in_specs=[
            pl.BlockSpec((blk, DIM), lambda i: (i, 0)),
            pl.BlockSpec((blk, DIM), lambda i: (i, 0)),
            pl.BlockSpec((blk, DIM), lambda i: (i, 0)),
            pl.BlockSpec((blk, DIM), lambda i: (i, 0)),
            pl.BlockSpec((blk, 1), lambda i: (i, 0)),
        ],
        out_specs=pl.BlockSpec((blk, DIM), lambda i: (i, 0)),
    )(emb_p, t1, t2, t3, dis2d)


def kernel(emb_weight, edge_index):
    ei = edge_index.astype(jnp.int32)
    pad = EPAD - N_EDGES
    row = jnp.concatenate([ei[0], jnp.zeros((pad,), jnp.int32)])
    col = jnp.concatenate([ei[1], jnp.full((pad,), NPAD, jnp.int32)])
    row_t = row.reshape(NS, NB, BATCH)
    col_t = col.reshape(NS, NB, BATCH)
    emb_p = jnp.pad(emb_weight, ((0, NPAD - N_NODES), (0, 0)))

    dis, t0, rowc, locc, cnts = _prep(col_t, row_t, emb_p)
    rowc4 = rowc.reshape(NC, NS, NB, BATCH)
    locc4 = locc.reshape(NC, NS, NB, BATCH)
    t1 = _layer(t0, rowc4, locc4, cnts, dis)
    t2 = _layer(t1, rowc4, locc4, cnts, dis)
    t3 = _layer(t2, rowc4, locc4, cnts, dis)
    out = _mean(emb_p, t1, t2, t3, dis.reshape(NPAD, 1))
    return out[:N_NODES]
